# baseline (speedup 1.0000x reference)
import functools
import math

import numpy as np
import jax
import jax.numpy as jnp
from jax import lax
from jax.experimental import pallas as pl
from jax.experimental.pallas import tpu as pltpu
from jax.experimental.pallas import tpu_sc as plsc

NUM_EXPERTS = 8
HIDDEN = 2048
INTER = 2048
TOKENS = 2048
CAPACITY = 640
NSLOTS = NUM_EXPERTS * CAPACITY
DROWS = CAPACITY * (NUM_EXPERTS + 1)
DUMP = NSLOTS
LANES = 128

NW = 32
TOK_PER_W = TOKENS // NW
DC = 16
NDC = TOK_PER_W // DC
CC = 8
NCC = TOK_PER_W // CC


def _router_body(x_ref, g_ref, s1_ref, s2_ref, w1_ref, w2_ref):
    x = x_ref[...]
    g = g_ref[...]
    logits = jnp.dot(x, g, preferred_element_type=jnp.float32)

    cols = lax.broadcasted_iota(jnp.int32, (TOKENS, LANES), 1)
    valid = cols < NUM_EXPERTS
    neg = jnp.float32(-1e30)
    lg = jnp.where(valid, logits, neg)
    m = jnp.max(lg, axis=1, keepdims=True)
    p = jnp.exp(lg - m)
    p = jnp.where(valid, p, 0.0)
    probs = p / jnp.sum(p, axis=1, keepdims=True)

    m1 = jnp.max(probs, axis=1, keepdims=True)
    is1 = valid & (probs == m1)
    c1 = jnp.min(jnp.where(is1, cols, LANES), axis=1, keepdims=True)
    mask1 = (cols == c1).astype(jnp.float32)
    p2 = jnp.where(mask1 > 0, -1.0, jnp.where(valid, probs, -1.0))
    m2 = jnp.max(p2, axis=1, keepdims=True)
    is2 = valid & (p2 == m2)
    c2 = jnp.min(jnp.where(is2, cols, LANES), axis=1, keepdims=True)
    mask2 = (cols == c2).astype(jnp.float32)

    ri = lax.broadcasted_iota(jnp.int32, (TOKENS, TOKENS), 0)
    ci = lax.broadcasted_iota(jnp.int32, (TOKENS, TOKENS), 1)
    ltri = (ri >= ci).astype(jnp.float32)
    rank1 = jnp.dot(ltri, mask1, preferred_element_type=jnp.float32) - 1.0
    rank2 = jnp.dot(ltri, mask2, preferred_element_type=jnp.float32) - 1.0
    count1 = jnp.sum(mask1, axis=0, keepdims=True)
    rank2 = rank2 + count1

    cap = jnp.float32(CAPACITY)
    keep1 = mask1 * (rank1 < cap).astype(jnp.float32)
    keep2 = mask2 * (rank2 < cap).astype(jnp.float32)
    r1 = jnp.sum(keep1 * rank1, axis=1, keepdims=True)
    r2 = jnp.sum(keep2 * rank2, axis=1, keepdims=True)
    w1 = jnp.sum(keep1 * probs, axis=1, keepdims=True)
    w2 = jnp.sum(keep2 * probs, axis=1, keepdims=True)
    kept1 = jnp.sum(keep1, axis=1, keepdims=True) > 0
    kept2 = jnp.sum(keep2, axis=1, keepdims=True) > 0

    slot1 = jnp.where(kept1, c1 * CAPACITY + r1.astype(jnp.int32), DUMP)
    slot2 = jnp.where(kept2, c2 * CAPACITY + r2.astype(jnp.int32), DUMP)
    w1 = jnp.where(kept1, w1, 0.0)
    w2 = jnp.where(kept2, w2, 0.0)

    ones = jnp.ones((1, LANES), jnp.float32)
    iones = jnp.ones((1, LANES), jnp.int32)
    s1_ref[...] = slot1 * iones
    s2_ref[...] = slot2 * iones
    w1_ref[...] = w1 * ones
    w2_ref[...] = w2 * ones


def _router(x, gpad):
    return pl.pallas_call(
        _router_body,
        out_shape=[
            jax.ShapeDtypeStruct((TOKENS, LANES), jnp.int32),
            jax.ShapeDtypeStruct((TOKENS, LANES), jnp.int32),
            jax.ShapeDtypeStruct((TOKENS, LANES), jnp.float32),
            jax.ShapeDtypeStruct((TOKENS, LANES), jnp.float32),
        ],
    )(x, gpad)


def _dispatch_body(s1_hbm, s2_hbm, tok_hbm, d_hbm,
                   idx1_v, idx2_v, bufs, seml0, seml1, sems0, sems1):
    cid = lax.axis_index("c")
    sid = lax.axis_index("s")
    wid = cid * 16 + sid
    tb = wid * TOK_PER_W

    pltpu.sync_copy(s1_hbm.at[wid], idx1_v)
    pltpu.sync_copy(s2_hbm.at[wid], idx2_v)

    seml = [seml0, seml1]
    sems = [sems0, sems1]
    loads = {}
    scats = {}

    def load(j):
        loads[j] = pltpu.async_copy(
            tok_hbm.at[pl.ds(tb + j * DC, DC)], bufs.at[j % 2], seml[j % 2])

    load(0)
    load(1)
    for j in range(NDC):
        b = j % 2
        if j >= 2:
            d1, d2 = scats[j - 2]
            d1.wait()
            d2.wait()
            load(j)
        loads[j].wait()
        d1 = pltpu.async_copy(bufs.at[b], d_hbm.at[idx1_v.at[j]], sems[b])
        d2 = pltpu.async_copy(bufs.at[b], d_hbm.at[idx2_v.at[j]], sems[b])
        scats[j] = (d1, d2)
    for j in (NDC - 2, NDC - 1):
        d1, d2 = scats[j]
        d1.wait()
        d2.wait()


def _dispatch(s1r, s2r, x):
    mesh = plsc.VectorSubcoreMesh(core_axis_name="c", subcore_axis_name="s")
    return pl.kernel(
        _dispatch_body,
        out_type=jax.ShapeDtypeStruct((DROWS, HIDDEN), jnp.float32),
        mesh=mesh,
        compiler_params=pltpu.CompilerParams(needs_layout_passes=False),
        scratch_types=[
            pltpu.VMEM((NDC, DC), jnp.int32),
            pltpu.VMEM((NDC, DC), jnp.int32),
            pltpu.VMEM((2, DC, HIDDEN), jnp.float32),
            pltpu.SemaphoreType.DMA,
            pltpu.SemaphoreType.DMA,
            pltpu.SemaphoreType.DMA,
            pltpu.SemaphoreType.DMA,
        ],
    )(s1r, s2r, x)


def _ffn_body(d_ref, wi_ref, wo_ref, o_ref):
    ki = pl.program_id(1)
    d = d_ref[...].astype(jnp.bfloat16)
    h = jnp.dot(d, wi_ref[0], preferred_element_type=jnp.float32)
    h = h * 0.5 * (1.0 + lax.erf(h * np.float32(1.0 / math.sqrt(2.0))))
    acc = jnp.dot(h.astype(jnp.bfloat16), wo_ref[0],
                  preferred_element_type=jnp.float32)

    @pl.when(ki == 0)
    def _():
        o_ref[...] = acc

    @pl.when(ki != 0)
    def _():
        o_ref[...] = o_ref[...] + acc


def _ffn(d, wi, wo, k_tiles=4):
    kt = INTER // k_tiles
    return pl.pallas_call(
        _ffn_body,
        grid=(NUM_EXPERTS, k_tiles),
        in_specs=[
            pl.BlockSpec((CAPACITY, HIDDEN), lambda e, k: (e, 0)),
            pl.BlockSpec((1, HIDDEN, kt), lambda e, k: (e, 0, k)),
            pl.BlockSpec((1, kt, HIDDEN), lambda e, k: (e, k, 0)),
        ],
        out_specs=pl.BlockSpec((CAPACITY, HIDDEN), lambda e, k: (e, 0)),
        out_shape=jax.ShapeDtypeStruct((NSLOTS, HIDDEN), jnp.float32),
    )(d, wi, wo)


def _combine_body(o_hbm, s12_hbm, w1_hbm, w2_hbm, out_hbm,
                  idx_v, wall1, wall2, rows, outb,
                  semg0, semg1, semw0, semw1):
    cid = lax.axis_index("c")
    sid = lax.axis_index("s")
    wid = cid * 16 + sid
    tb = wid * TOK_PER_W

    pltpu.sync_copy(s12_hbm.at[wid], idx_v)
    pltpu.sync_copy(w1_hbm.at[pl.ds(tb, TOK_PER_W)], wall1)
    pltpu.sync_copy(w2_hbm.at[pl.ds(tb, TOK_PER_W)], wall2)

    semg = [semg0, semg1]
    semw = [semw0, semw1]
    gath = {}
    writes = {}

    def fire(j):
        b = j % 2
        gath[j] = pltpu.async_copy(
            o_hbm.at[idx_v.at[j]], rows.at[b], semg[b])

    fire(0)
    fire(1)
    zero = jnp.zeros((16,), jnp.float32)
    for j in range(NCC):
        b = j % 2
        gath[j].wait()
        if j >= 2:
            writes[j - 2].wait()

        def tok_body(i, _):
            w1v = wall1[j * CC + i]
            w2v = wall2[j * CC + i]
            m1 = w1v != 0.0
            m2 = w2v != 0.0

            def vec_body(v, _):
                for u in range(8):
                    off = (v * 8 + u) * 16
                    a = rows[b, i, pl.ds(off, 16)]
                    c = rows[b, i + CC, pl.ds(off, 16)]
                    r = jnp.where(m1, a * w1v, zero) + \
                        jnp.where(m2, c * w2v, zero)
                    outb[b, i, pl.ds(off, 16)] = r
                return 0
            lax.fori_loop(0, HIDDEN // 128, vec_body, 0)
            return 0
        lax.fori_loop(0, CC, tok_body, 0)
        writes[j] = pltpu.async_copy(
            outb.at[b], out_hbm.at[pl.ds(tb + j * CC, CC)], semw[b])
        if j + 2 < NCC:
            fire(j + 2)
    writes[NCC - 2].wait()
    writes[NCC - 1].wait()


def _combine(o, s12, w1s, w2s):
    mesh = plsc.VectorSubcoreMesh(core_axis_name="c", subcore_axis_name="s")
    return pl.kernel(
        _combine_body,
        out_type=jax.ShapeDtypeStruct((TOKENS, HIDDEN), jnp.float32),
        mesh=mesh,
        scratch_types=[
            pltpu.VMEM((NCC, 2 * CC), jnp.int32),
            pltpu.VMEM((TOK_PER_W, 16), jnp.float32),
            pltpu.VMEM((TOK_PER_W, 16), jnp.float32),
            pltpu.VMEM((2, 2 * CC, HIDDEN), jnp.float32),
            pltpu.VMEM((2, CC, HIDDEN), jnp.float32),
            pltpu.SemaphoreType.DMA,
            pltpu.SemaphoreType.DMA,
            pltpu.SemaphoreType.DMA,
            pltpu.SemaphoreType.DMA,
        ],
    )(o, s12, w1s, w2s)


def kernel(inputs, gate_weight, wi, wo):
    x = inputs.reshape(TOKENS, HIDDEN).astype(jnp.float32)
    gpad = jnp.zeros((HIDDEN, LANES), jnp.float32)
    gpad = gpad.at[:, :NUM_EXPERTS].set(gate_weight.astype(jnp.float32).T)

    s1b, s2b, w1b, w2b = _router(x, gpad)
    s1 = s1b[:, 0]
    s2 = s2b[:, 0]
    w1s = w1b[:, :16]
    w2s = w2b[:, :16]

    d = _dispatch(s1.reshape(NW, NDC, DC), s2.reshape(NW, NDC, DC), x)
    o = _ffn(d, wi.astype(jnp.bfloat16), wo.astype(jnp.bfloat16))

    s1c = jnp.minimum(s1, NSLOTS - 1).reshape(NW, NCC, CC)
    s2c = jnp.minimum(s2, NSLOTS - 1).reshape(NW, NCC, CC)
    s12 = jnp.concatenate([s1c, s2c], axis=-1)
    out = _combine(o, s12, w1s, w2s)
    return out.reshape(inputs.shape)


# speedup vs baseline: 1.4004x; 1.4004x over previous
import functools
import math

import numpy as np
import jax
import jax.numpy as jnp
from jax import lax
from jax.experimental import pallas as pl
from jax.experimental.pallas import tpu as pltpu
from jax.experimental.pallas import tpu_sc as plsc

NUM_EXPERTS = 8
HIDDEN = 2048
INTER = 2048
TOKENS = 2048
CAPACITY = 640
NSLOTS = NUM_EXPERTS * CAPACITY
DROWS = CAPACITY * (NUM_EXPERTS + 1)
DUMP = NSLOTS
LANES = 128

NW = 32
TOK_PER_W = TOKENS // NW
DC = 16
NDC = TOK_PER_W // DC
CC = 8
NCC = TOK_PER_W // CC


def _router_body(x_ref, g_ref, s1_ref, s2_ref, w1_ref, w2_ref):
    x = x_ref[...]
    g = g_ref[...]
    logits = jnp.dot(x, g, preferred_element_type=jnp.float32)

    cols = lax.broadcasted_iota(jnp.int32, (TOKENS, LANES), 1)
    valid = cols < NUM_EXPERTS
    neg = jnp.float32(-1e30)
    lg = jnp.where(valid, logits, neg)
    m = jnp.max(lg, axis=1, keepdims=True)
    p = jnp.exp(lg - m)
    p = jnp.where(valid, p, 0.0)
    probs = p / jnp.sum(p, axis=1, keepdims=True)

    m1 = jnp.max(probs, axis=1, keepdims=True)
    is1 = valid & (probs == m1)
    c1 = jnp.min(jnp.where(is1, cols, LANES), axis=1, keepdims=True)
    mask1 = (cols == c1).astype(jnp.float32)
    p2 = jnp.where(mask1 > 0, -1.0, jnp.where(valid, probs, -1.0))
    m2 = jnp.max(p2, axis=1, keepdims=True)
    is2 = valid & (p2 == m2)
    c2 = jnp.min(jnp.where(is2, cols, LANES), axis=1, keepdims=True)
    mask2 = (cols == c2).astype(jnp.float32)

    ri = lax.broadcasted_iota(jnp.int32, (TOKENS, TOKENS), 0)
    ci = lax.broadcasted_iota(jnp.int32, (TOKENS, TOKENS), 1)
    ltri = (ri >= ci).astype(jnp.float32)
    rank1 = jnp.dot(ltri, mask1, preferred_element_type=jnp.float32) - 1.0
    rank2 = jnp.dot(ltri, mask2, preferred_element_type=jnp.float32) - 1.0
    count1 = jnp.sum(mask1, axis=0, keepdims=True)
    rank2 = rank2 + count1

    cap = jnp.float32(CAPACITY)
    keep1 = mask1 * (rank1 < cap).astype(jnp.float32)
    keep2 = mask2 * (rank2 < cap).astype(jnp.float32)
    r1 = jnp.sum(keep1 * rank1, axis=1, keepdims=True)
    r2 = jnp.sum(keep2 * rank2, axis=1, keepdims=True)
    w1 = jnp.sum(keep1 * probs, axis=1, keepdims=True)
    w2 = jnp.sum(keep2 * probs, axis=1, keepdims=True)
    kept1 = jnp.sum(keep1, axis=1, keepdims=True) > 0
    kept2 = jnp.sum(keep2, axis=1, keepdims=True) > 0

    slot1 = jnp.where(kept1, c1 * CAPACITY + r1.astype(jnp.int32), DUMP)
    slot2 = jnp.where(kept2, c2 * CAPACITY + r2.astype(jnp.int32), DUMP)
    w1 = jnp.where(kept1, w1, 0.0)
    w2 = jnp.where(kept2, w2, 0.0)

    ones = jnp.ones((1, LANES), jnp.float32)
    iones = jnp.ones((1, LANES), jnp.int32)
    s1_ref[...] = slot1 * iones
    s2_ref[...] = slot2 * iones
    w1_ref[...] = w1 * ones
    w2_ref[...] = w2 * ones


def _router(x, gpad):
    return pl.pallas_call(
        _router_body,
        out_shape=[
            jax.ShapeDtypeStruct((TOKENS, LANES), jnp.int32),
            jax.ShapeDtypeStruct((TOKENS, LANES), jnp.int32),
            jax.ShapeDtypeStruct((TOKENS, LANES), jnp.float32),
            jax.ShapeDtypeStruct((TOKENS, LANES), jnp.float32),
        ],
    )(x, gpad)


def _dispatch_body(s1_hbm, s2_hbm, tok_hbm, d_hbm,
                   idx1_v, idx2_v, bufs, seml0, seml1, sems0, sems1):
    cid = lax.axis_index("c")
    sid = lax.axis_index("s")
    wid = cid * 16 + sid
    tb = wid * TOK_PER_W

    pltpu.sync_copy(s1_hbm.at[wid], idx1_v)
    pltpu.sync_copy(s2_hbm.at[wid], idx2_v)

    seml = [seml0, seml1]
    sems = [sems0, sems1]
    loads = {}
    scats = {}

    def load(j):
        loads[j] = pltpu.async_copy(
            tok_hbm.at[pl.ds(tb + j * DC, DC)], bufs.at[j % 2], seml[j % 2])

    load(0)
    load(1)
    for j in range(NDC):
        b = j % 2
        if j >= 2:
            d1, d2 = scats[j - 2]
            d1.wait()
            d2.wait()
            load(j)
        loads[j].wait()
        d1 = pltpu.async_copy(bufs.at[b], d_hbm.at[idx1_v.at[j]], sems[b])
        d2 = pltpu.async_copy(bufs.at[b], d_hbm.at[idx2_v.at[j]], sems[b])
        scats[j] = (d1, d2)
    for j in (NDC - 2, NDC - 1):
        d1, d2 = scats[j]
        d1.wait()
        d2.wait()


def _dispatch(s1r, s2r, x):
    mesh = plsc.VectorSubcoreMesh(core_axis_name="c", subcore_axis_name="s")
    return pl.kernel(
        _dispatch_body,
        out_type=jax.ShapeDtypeStruct((DROWS, HIDDEN), jnp.float32),
        mesh=mesh,
        compiler_params=pltpu.CompilerParams(needs_layout_passes=False),
        scratch_types=[
            pltpu.VMEM((NDC, DC), jnp.int32),
            pltpu.VMEM((NDC, DC), jnp.int32),
            pltpu.VMEM((2, DC, HIDDEN), jnp.float32),
            pltpu.SemaphoreType.DMA,
            pltpu.SemaphoreType.DMA,
            pltpu.SemaphoreType.DMA,
            pltpu.SemaphoreType.DMA,
        ],
    )(s1r, s2r, x)


def _ffn_body(d_ref, wi_ref, wo_ref, o_ref):
    ki = pl.program_id(1)
    d = d_ref[...].astype(jnp.bfloat16)
    h = jnp.dot(d, wi_ref[0].astype(jnp.bfloat16),
                preferred_element_type=jnp.float32)
    h = h * 0.5 * (1.0 + lax.erf(h * np.float32(1.0 / math.sqrt(2.0))))
    acc = jnp.dot(h.astype(jnp.bfloat16), wo_ref[0].astype(jnp.bfloat16),
                  preferred_element_type=jnp.float32)

    @pl.when(ki == 0)
    def _():
        o_ref[...] = acc

    @pl.when(ki != 0)
    def _():
        o_ref[...] = o_ref[...] + acc


def _ffn(d, wi, wo, k_tiles=4):
    kt = INTER // k_tiles
    return pl.pallas_call(
        _ffn_body,
        grid=(NUM_EXPERTS, k_tiles),
        in_specs=[
            pl.BlockSpec((CAPACITY, HIDDEN), lambda e, k: (e, 0)),
            pl.BlockSpec((1, HIDDEN, kt), lambda e, k: (e, 0, k)),
            pl.BlockSpec((1, kt, HIDDEN), lambda e, k: (e, k, 0)),
        ],
        out_specs=pl.BlockSpec((CAPACITY, HIDDEN), lambda e, k: (e, 0)),
        out_shape=jax.ShapeDtypeStruct((NSLOTS, HIDDEN), jnp.float32),
    )(d, wi, wo)


def _combine_body(o_hbm, s12_hbm, w1_hbm, w2_hbm, out_hbm,
                  idx_v, wall1, wall2, rows, outb,
                  semg0, semg1, semw0, semw1):
    cid = lax.axis_index("c")
    sid = lax.axis_index("s")
    wid = cid * 16 + sid
    tb = wid * TOK_PER_W

    pltpu.sync_copy(s12_hbm.at[wid], idx_v)
    pltpu.sync_copy(w1_hbm.at[pl.ds(tb, TOK_PER_W)], wall1)
    pltpu.sync_copy(w2_hbm.at[pl.ds(tb, TOK_PER_W)], wall2)

    semg = [semg0, semg1]
    semw = [semw0, semw1]
    gath = {}
    writes = {}

    def fire(j):
        b = j % 2
        gath[j] = pltpu.async_copy(
            o_hbm.at[idx_v.at[j]], rows.at[b], semg[b])

    fire(0)
    fire(1)
    zero = jnp.zeros((16,), jnp.float32)
    for j in range(NCC):
        b = j % 2
        gath[j].wait()
        if j >= 2:
            writes[j - 2].wait()

        def tok_body(i, _):
            w1v = wall1[j * CC + i]
            w2v = wall2[j * CC + i]
            m1 = w1v != 0.0
            m2 = w2v != 0.0

            def vec_body(v, _):
                for u in range(8):
                    off = (v * 8 + u) * 16
                    a = rows[b, i, pl.ds(off, 16)]
                    c = rows[b, i + CC, pl.ds(off, 16)]
                    r = jnp.where(m1, a * w1v, zero) + \
                        jnp.where(m2, c * w2v, zero)
                    outb[b, i, pl.ds(off, 16)] = r
                return 0
            lax.fori_loop(0, HIDDEN // 128, vec_body, 0)
            return 0
        lax.fori_loop(0, CC, tok_body, 0)
        writes[j] = pltpu.async_copy(
            outb.at[b], out_hbm.at[pl.ds(tb + j * CC, CC)], semw[b])
        if j + 2 < NCC:
            fire(j + 2)
    writes[NCC - 2].wait()
    writes[NCC - 1].wait()


def _combine(o, s12, w1s, w2s):
    mesh = plsc.VectorSubcoreMesh(core_axis_name="c", subcore_axis_name="s")
    return pl.kernel(
        _combine_body,
        out_type=jax.ShapeDtypeStruct((TOKENS, HIDDEN), jnp.float32),
        mesh=mesh,
        scratch_types=[
            pltpu.VMEM((NCC, 2 * CC), jnp.int32),
            pltpu.VMEM((TOK_PER_W, 16), jnp.float32),
            pltpu.VMEM((TOK_PER_W, 16), jnp.float32),
            pltpu.VMEM((2, 2 * CC, HIDDEN), jnp.float32),
            pltpu.VMEM((2, CC, HIDDEN), jnp.float32),
            pltpu.SemaphoreType.DMA,
            pltpu.SemaphoreType.DMA,
            pltpu.SemaphoreType.DMA,
            pltpu.SemaphoreType.DMA,
        ],
    )(o, s12, w1s, w2s)


def kernel(inputs, gate_weight, wi, wo):
    x = inputs.reshape(TOKENS, HIDDEN).astype(jnp.float32)
    gpad = jnp.zeros((HIDDEN, LANES), jnp.float32)
    gpad = gpad.at[:, :NUM_EXPERTS].set(gate_weight.astype(jnp.float32).T)

    s1b, s2b, w1b, w2b = _router(x, gpad)
    s1 = s1b[:, 0]
    s2 = s2b[:, 0]
    w1s = w1b[:, :16]
    w2s = w2b[:, :16]

    d = _dispatch(s1.reshape(NW, NDC, DC), s2.reshape(NW, NDC, DC), x)
    o = _ffn(d, wi, wo)

    s1c = jnp.minimum(s1, NSLOTS - 1).reshape(NW, NCC, CC)
    s2c = jnp.minimum(s2, NSLOTS - 1).reshape(NW, NCC, CC)
    s12 = jnp.concatenate([s1c, s2c], axis=-1)
    out = _combine(o, s12, w1s, w2s)
    return out.reshape(inputs.shape)


# speedup vs baseline: 1.5248x; 1.0888x over previous
import functools
import math

import numpy as np
import jax
import jax.numpy as jnp
from jax import lax
from jax.experimental import pallas as pl
from jax.experimental.pallas import tpu as pltpu
from jax.experimental.pallas import tpu_sc as plsc

NUM_EXPERTS = 8
HIDDEN = 2048
INTER = 2048
TOKENS = 2048
CAPACITY = 640
NSLOTS = NUM_EXPERTS * CAPACITY
DROWS = CAPACITY * (NUM_EXPERTS + 1)
DUMP = NSLOTS
LANES = 128

NW = 32
TOK_PER_W = TOKENS // NW
DC = 16
NDC = TOK_PER_W // DC
CC = 8
NCC = TOK_PER_W // CC


def _router_body(x_ref, g_ref, s1_ref, s2_ref, w1_ref, w2_ref):
    x = x_ref[...]
    g = g_ref[...]
    logits = lax.dot_general(
        x, g, (((1,), (1,)), ((), ())),
        preferred_element_type=jnp.float32)

    E = NUM_EXPERTS
    cols = lax.broadcasted_iota(jnp.int32, (TOKENS, E), 1)
    m = jnp.max(logits, axis=1, keepdims=True)
    p = jnp.exp(logits - m)
    probs = p / jnp.sum(p, axis=1, keepdims=True)

    m1 = jnp.max(probs, axis=1, keepdims=True)
    is1 = probs == m1
    c1 = jnp.min(jnp.where(is1, cols, E), axis=1, keepdims=True)
    mask1 = (cols == c1).astype(jnp.float32)
    p2 = jnp.where(mask1 > 0, -1.0, probs)
    m2 = jnp.max(p2, axis=1, keepdims=True)
    is2 = p2 == m2
    c2 = jnp.min(jnp.where(is2, cols, E), axis=1, keepdims=True)
    mask2 = (cols == c2).astype(jnp.float32)

    ri = lax.broadcasted_iota(jnp.int32, (TOKENS, TOKENS), 0)
    ci = lax.broadcasted_iota(jnp.int32, (TOKENS, TOKENS), 1)
    ltri = (ri >= ci).astype(jnp.float32)
    rank1 = jnp.dot(ltri, mask1, preferred_element_type=jnp.float32) - 1.0
    rank2 = jnp.dot(ltri, mask2, preferred_element_type=jnp.float32) - 1.0
    count1 = jnp.sum(mask1, axis=0, keepdims=True)
    rank2 = rank2 + count1

    cap = jnp.float32(CAPACITY)
    keep1 = mask1 * (rank1 < cap).astype(jnp.float32)
    keep2 = mask2 * (rank2 < cap).astype(jnp.float32)
    r1 = jnp.sum(keep1 * rank1, axis=1, keepdims=True)
    r2 = jnp.sum(keep2 * rank2, axis=1, keepdims=True)
    w1 = jnp.sum(keep1 * probs, axis=1, keepdims=True)
    w2 = jnp.sum(keep2 * probs, axis=1, keepdims=True)
    kept1 = jnp.sum(keep1, axis=1, keepdims=True) > 0
    kept2 = jnp.sum(keep2, axis=1, keepdims=True) > 0

    slot1 = jnp.where(kept1, c1 * CAPACITY + r1.astype(jnp.int32), DUMP)
    slot2 = jnp.where(kept2, c2 * CAPACITY + r2.astype(jnp.int32), DUMP)
    w1 = jnp.where(kept1, w1, 0.0)
    w2 = jnp.where(kept2, w2, 0.0)

    s1_ref[...] = slot1
    s2_ref[...] = slot2
    w1_ref[...] = jnp.broadcast_to(w1, (TOKENS, 16))
    w2_ref[...] = jnp.broadcast_to(w2, (TOKENS, 16))


def _router(x, g):
    return pl.pallas_call(
        _router_body,
        out_shape=[
            jax.ShapeDtypeStruct((TOKENS, 1), jnp.int32),
            jax.ShapeDtypeStruct((TOKENS, 1), jnp.int32),
            jax.ShapeDtypeStruct((TOKENS, 16), jnp.float32),
            jax.ShapeDtypeStruct((TOKENS, 16), jnp.float32),
        ],
    )(x, g)


def _dispatch_body(s1_hbm, s2_hbm, tok_hbm, d_hbm,
                   idx1_v, idx2_v, bufs, seml0, seml1, sems0, sems1):
    cid = lax.axis_index("c")
    sid = lax.axis_index("s")
    wid = cid * 16 + sid
    tb = wid * TOK_PER_W

    pltpu.sync_copy(s1_hbm.at[wid], idx1_v)
    pltpu.sync_copy(s2_hbm.at[wid], idx2_v)

    seml = [seml0, seml1]
    sems = [sems0, sems1]
    loads = {}
    scats = {}

    def load(j):
        loads[j] = pltpu.async_copy(
            tok_hbm.at[pl.ds(tb + j * DC, DC)], bufs.at[j % 2], seml[j % 2])

    load(0)
    load(1)
    for j in range(NDC):
        b = j % 2
        if j >= 2:
            d1, d2 = scats[j - 2]
            d1.wait()
            d2.wait()
            load(j)
        loads[j].wait()
        d1 = pltpu.async_copy(bufs.at[b], d_hbm.at[idx1_v.at[j]], sems[b])
        d2 = pltpu.async_copy(bufs.at[b], d_hbm.at[idx2_v.at[j]], sems[b])
        scats[j] = (d1, d2)
    for j in (NDC - 2, NDC - 1):
        d1, d2 = scats[j]
        d1.wait()
        d2.wait()


def _dispatch(s1r, s2r, x):
    mesh = plsc.VectorSubcoreMesh(core_axis_name="c", subcore_axis_name="s")
    return pl.kernel(
        _dispatch_body,
        out_type=jax.ShapeDtypeStruct((DROWS, HIDDEN), jnp.float32),
        mesh=mesh,
        compiler_params=pltpu.CompilerParams(needs_layout_passes=False),
        scratch_types=[
            pltpu.VMEM((NDC, DC), jnp.int32),
            pltpu.VMEM((NDC, DC), jnp.int32),
            pltpu.VMEM((2, DC, HIDDEN), jnp.float32),
            pltpu.SemaphoreType.DMA,
            pltpu.SemaphoreType.DMA,
            pltpu.SemaphoreType.DMA,
            pltpu.SemaphoreType.DMA,
        ],
    )(s1r, s2r, x)


def _ffn_body(d_ref, wi_ref, wo_ref, o_ref):
    ki = pl.program_id(1)
    d = d_ref[...].astype(jnp.bfloat16)
    h = jnp.dot(d, wi_ref[0].astype(jnp.bfloat16),
                preferred_element_type=jnp.float32)
    h = h * 0.5 * (1.0 + lax.erf(h * np.float32(1.0 / math.sqrt(2.0))))
    acc = jnp.dot(h.astype(jnp.bfloat16), wo_ref[0].astype(jnp.bfloat16),
                  preferred_element_type=jnp.float32)

    @pl.when(ki == 0)
    def _():
        o_ref[...] = acc

    @pl.when(ki != 0)
    def _():
        o_ref[...] = o_ref[...] + acc


def _ffn(d, wi, wo, k_tiles=2):
    kt = INTER // k_tiles
    return pl.pallas_call(
        _ffn_body,
        grid=(NUM_EXPERTS, k_tiles),
        in_specs=[
            pl.BlockSpec((CAPACITY, HIDDEN), lambda e, k: (e, 0)),
            pl.BlockSpec((1, HIDDEN, kt), lambda e, k: (e, 0, k)),
            pl.BlockSpec((1, kt, HIDDEN), lambda e, k: (e, k, 0)),
        ],
        out_specs=pl.BlockSpec((CAPACITY, HIDDEN), lambda e, k: (e, 0)),
        out_shape=jax.ShapeDtypeStruct((NSLOTS, HIDDEN), jnp.float32),
    )(d, wi, wo)


def _combine_body(o_hbm, s12_hbm, w1_hbm, w2_hbm, out_hbm,
                  idx_v, wall1, wall2, rows, outb,
                  semg0, semg1, semw0, semw1):
    cid = lax.axis_index("c")
    sid = lax.axis_index("s")
    wid = cid * 16 + sid
    tb = wid * TOK_PER_W

    pltpu.sync_copy(s12_hbm.at[wid], idx_v)
    pltpu.sync_copy(w1_hbm.at[pl.ds(tb, TOK_PER_W)], wall1)
    pltpu.sync_copy(w2_hbm.at[pl.ds(tb, TOK_PER_W)], wall2)

    semg = [semg0, semg1]
    semw = [semw0, semw1]
    gath = {}
    writes = {}

    def fire(j):
        b = j % 2
        gath[j] = pltpu.async_copy(
            o_hbm.at[idx_v.at[j]], rows.at[b], semg[b])

    fire(0)
    fire(1)
    zero = jnp.zeros((16,), jnp.float32)
    for j in range(NCC):
        b = j % 2
        gath[j].wait()
        if j >= 2:
            writes[j - 2].wait()

        def tok_body(i, _):
            w1v = wall1[j * CC + i]
            w2v = wall2[j * CC + i]
            m1 = w1v != 0.0
            m2 = w2v != 0.0

            def vec_body(v, _):
                for u in range(8):
                    off = (v * 8 + u) * 16
                    a = rows[b, i, pl.ds(off, 16)]
                    c = rows[b, i + CC, pl.ds(off, 16)]
                    r = jnp.where(m1, a * w1v, zero) + \
                        jnp.where(m2, c * w2v, zero)
                    outb[b, i, pl.ds(off, 16)] = r
                return 0
            lax.fori_loop(0, HIDDEN // 128, vec_body, 0)
            return 0
        lax.fori_loop(0, CC, tok_body, 0)
        writes[j] = pltpu.async_copy(
            outb.at[b], out_hbm.at[pl.ds(tb + j * CC, CC)], semw[b])
        if j + 2 < NCC:
            fire(j + 2)
    writes[NCC - 2].wait()
    writes[NCC - 1].wait()


def _combine(o, s12, w1s, w2s):
    mesh = plsc.VectorSubcoreMesh(core_axis_name="c", subcore_axis_name="s")
    return pl.kernel(
        _combine_body,
        out_type=jax.ShapeDtypeStruct((TOKENS, HIDDEN), jnp.float32),
        mesh=mesh,
        scratch_types=[
            pltpu.VMEM((NCC, 2 * CC), jnp.int32),
            pltpu.VMEM((TOK_PER_W, 16), jnp.float32),
            pltpu.VMEM((TOK_PER_W, 16), jnp.float32),
            pltpu.VMEM((2, 2 * CC, HIDDEN), jnp.float32),
            pltpu.VMEM((2, CC, HIDDEN), jnp.float32),
            pltpu.SemaphoreType.DMA,
            pltpu.SemaphoreType.DMA,
            pltpu.SemaphoreType.DMA,
            pltpu.SemaphoreType.DMA,
        ],
    )(o, s12, w1s, w2s)


def kernel(inputs, gate_weight, wi, wo):
    x = inputs.reshape(TOKENS, HIDDEN).astype(jnp.float32)

    s1b, s2b, w1s, w2s = _router(x, gate_weight.astype(jnp.float32))
    s1 = s1b.reshape(TOKENS)
    s2 = s2b.reshape(TOKENS)

    d = _dispatch(s1.reshape(NW, NDC, DC), s2.reshape(NW, NDC, DC), x)
    o = _ffn(d, wi, wo)

    s1c = jnp.minimum(s1, NSLOTS - 1).reshape(NW, NCC, CC)
    s2c = jnp.minimum(s2, NSLOTS - 1).reshape(NW, NCC, CC)
    s12 = jnp.concatenate([s1c, s2c], axis=-1)
    out = _combine(o, s12, w1s, w2s)
    return out.reshape(inputs.shape)
